# baseline (speedup 1.0000x reference)
import functools

import jax
import jax.numpy as jnp
from jax import lax
from jax.experimental import pallas as pl
from jax.experimental.pallas import tpu as pltpu
from jax.experimental.pallas import tpu_sc as plsc

L = 4
G = 10000
D = 128
OUT = 128
M = 32
E = 320000
CAT = OUT + 2 * M
LM = L * M

NS = 16
CHUNK = 128
NCHUNKS = E // CHUNK
CPT = NCHUNKS // NS
CEXTRA = NCHUNKS - CPT * NS
NRB = 3
NIB = 4
NUNROLL = 12
ROWS_PER_TILE = 624
ROWS_TAIL = G - NS * ROWS_PER_TILE

BG = 2000


def _proj_body(x_ref, w_ref, b_ref, pin_ref, pout_ref):
    w = w_ref[...]
    b = b_ref[...]
    pin_parts = []
    pout_parts = []
    for l in range(L):
        y = jnp.dot(x_ref[l], w, preferred_element_type=jnp.float32) + b
        pin_parts.append(y[:, :M])
        pout_parts.append(y[:, M:])
    pin_ref[...] = jnp.concatenate(pin_parts, axis=1)
    pout_ref[...] = jnp.concatenate(pout_parts, axis=1)


def _projections(x, w_agg, b_agg):
    grid = G // BG
    return pl.pallas_call(
        _proj_body,
        grid=(grid,),
        in_specs=[
            pl.BlockSpec((L, BG, D), lambda i: (0, i, 0)),
            pl.BlockSpec((D, 2 * M), lambda i: (0, 0)),
            pl.BlockSpec((1, 2 * M), lambda i: (0, 0)),
        ],
        out_specs=[
            pl.BlockSpec((BG, LM), lambda i: (i, 0)),
            pl.BlockSpec((BG, LM), lambda i: (i, 0)),
        ],
        out_shape=[
            jax.ShapeDtypeStruct((G, LM), jnp.float32),
            jax.ShapeDtypeStruct((G, LM), jnp.float32),
        ],
    )(x, w_agg, b_agg)


def _sc_scatter(p_in_r, p_out_r, idx_in, idx_out, zeros):
    mesh = plsc.VectorSubcoreMesh(core_axis_name="c", subcore_axis_name="s")

    @functools.partial(
        pl.kernel,
        mesh=mesh,
        out_type=[
            jax.ShapeDtypeStruct((G, LM), jnp.float32),
            jax.ShapeDtypeStruct((G, LM), jnp.float32),
        ],
        scratch_types=(
            [pltpu.VMEM_SHARED((G, LM), jnp.float32)]
            + [pltpu.VMEM((2, CHUNK), jnp.int32) for _ in range(NIB)]
            + [pltpu.VMEM((CHUNK, LM), jnp.float32) for _ in range(NRB)]
            + [pltpu.SemaphoreType.DMA for _ in range(NIB + 2 * NRB)]
        ),
    )
    def k(tab_in, tab_out, idx_i, idx_o, z, out_in, out_out, acc, *bufs):
        ibufs = bufs[:NIB]
        rbufs = bufs[NIB:NIB + NRB]
        si = bufs[NIB + NRB:2 * NIB + NRB]
        sg = bufs[2 * NIB + NRB:2 * NIB + 2 * NRB]
        ss = bufs[2 * NIB + 2 * NRB:]
        c = lax.axis_index("c")
        s = lax.axis_index("s")

        def run(tab, idx_hbm, out_hbm):
            my_rows = pl.ds(s * ROWS_PER_TILE, ROWS_PER_TILE)
            tail_rows = pl.ds(NS * ROWS_PER_TILE, ROWS_TAIL)
            pltpu.sync_copy(z, acc.at[my_rows])

            @pl.when(s == NS - 1)
            def _():
                pltpu.sync_copy(z.at[pl.ds(0, ROWS_TAIL)], acc.at[tail_rows])

            plsc.subcore_barrier()

            cb = s * CPT + jnp.minimum(s, CEXTRA)
            n_chunks = jnp.where(s < CEXTRA, CPT + 1, CPT)

            def fetch_idx(t, bi):
                base = (cb + t) * CHUNK
                pltpu.async_copy(idx_hbm.at[0, pl.ds(base, CHUNK)],
                                 ibufs[bi].at[0], si[bi])
                pltpu.async_copy(idx_hbm.at[1, pl.ds(base, CHUNK)],
                                 ibufs[bi].at[1], si[bi])

            def wait_idx(bi):
                pltpu.make_async_copy(idx_hbm.at[0, pl.ds(0, CHUNK)],
                                      ibufs[bi].at[0], si[bi]).wait()
                pltpu.make_async_copy(idx_hbm.at[1, pl.ds(0, CHUNK)],
                                      ibufs[bi].at[1], si[bi]).wait()

            def gather(bi, br):
                pltpu.async_copy(tab.at[ibufs[bi].at[1]], rbufs[br], sg[br])

            def wait_gather(bi, br):
                pltpu.make_async_copy(
                    tab.at[ibufs[bi].at[1]], rbufs[br], sg[br]).wait()

            def scatter(bi, br):
                pltpu.async_copy(
                    rbufs[br], acc.at[ibufs[bi].at[0]], ss[br], add=True)

            def wait_scatter(bi, br):
                pltpu.make_async_copy(
                    rbufs[br], acc.at[ibufs[bi].at[0]], ss[br]).wait()

            fetch_idx(jnp.int32(0), 0)
            fetch_idx(jnp.int32(1), 1)
            wait_idx(0)
            gather(0, 0)

            def body(o, carry):
                for off in range(NUNROLL):
                    t = o * NUNROLL + off

                    @pl.when(t >= 2)
                    def _():
                        wait_scatter((off + 2) % NIB, (off + 1) % NRB)

                    @pl.when(t + 2 < n_chunks)
                    def _():
                        fetch_idx(t + 2, (off + 2) % NIB)

                    @pl.when(t + 1 < n_chunks)
                    def _():
                        wait_idx((off + 1) % NIB)
                        gather((off + 1) % NIB, (off + 1) % NRB)

                    wait_gather(off % NIB, off % NRB)
                    scatter(off % NIB, off % NRB)
                return carry

            lax.fori_loop(0, CPT // NUNROLL, body, 0)

            @pl.when(s < CEXTRA)
            def _():
                wait_gather(0, 0)
                scatter(0, 0)

            wait_scatter(2, 1)
            wait_scatter(3, 2)

            @pl.when(s < CEXTRA)
            def _():
                wait_scatter(0, 0)

            plsc.subcore_barrier()
            pltpu.sync_copy(acc.at[my_rows], out_hbm.at[my_rows])

            @pl.when(s == NS - 1)
            def _():
                pltpu.sync_copy(acc.at[tail_rows], out_hbm.at[tail_rows])

        @pl.when(c == 0)
        def _():
            run(tab_in, idx_i, out_in)

        @pl.when(c == 1)
        def _():
            run(tab_out, idx_o, out_out)

    return k(p_in_r, p_out_r, idx_in, idx_out, zeros)


def _silu(v):
    return v / (1.0 + jnp.exp(-v))


def _mlp_body(x_ref, wl_ref, bl_ref, inw_ref, outw_ref,
              wm_ref, bm_ref, ln2s_ref, ln2b_ref, o_ref):
    wl = wl_ref[...]
    bl = bl_ref[...]
    ln2s = ln2s_ref[...]
    ln2b = ln2b_ref[...]
    wm = wm_ref[...]
    bm = bm_ref[...]
    inw = inw_ref[...]
    outw = outw_ref[...]
    for l in range(L):
        proj = jnp.dot(x_ref[l], wl, preferred_element_type=jnp.float32) + bl
        cat = jnp.concatenate(
            [proj, inw[:, l * M:(l + 1) * M], outw[:, l * M:(l + 1) * M]],
            axis=1)
        h = _silu(cat)
        mu = jnp.mean(h, axis=1, keepdims=True)
        msq = jnp.mean(h * h, axis=1, keepdims=True)
        h = (h - mu) * lax.rsqrt(msq - mu * mu + 1e-5)
        h = jnp.dot(h, wm, preferred_element_type=jnp.float32) + bm
        h = _silu(h)
        mu = jnp.mean(h, axis=1, keepdims=True)
        msq = jnp.mean(h * h, axis=1, keepdims=True)
        o_ref[l] = (h - mu) * lax.rsqrt(msq - mu * mu + 1e-5) * ln2s + ln2b


def _mlp(x, w_lin, b_lin, in_w, out_w, w_mlp, b_mlp, ln2s, ln2b):
    grid = G // BG
    return pl.pallas_call(
        _mlp_body,
        grid=(grid,),
        in_specs=[
            pl.BlockSpec((L, BG, D), lambda i: (0, i, 0)),
            pl.BlockSpec((D, OUT), lambda i: (0, 0)),
            pl.BlockSpec((1, OUT), lambda i: (0, 0)),
            pl.BlockSpec((BG, LM), lambda i: (i, 0)),
            pl.BlockSpec((BG, LM), lambda i: (i, 0)),
            pl.BlockSpec((CAT, OUT), lambda i: (0, 0)),
            pl.BlockSpec((1, OUT), lambda i: (0, 0)),
            pl.BlockSpec((1, OUT), lambda i: (0, 0)),
            pl.BlockSpec((1, OUT), lambda i: (0, 0)),
        ],
        out_specs=pl.BlockSpec((L, BG, OUT), lambda i: (0, i, 0)),
        out_shape=jax.ShapeDtypeStruct((L, G, OUT), jnp.float32),
    )(x, w_lin, b_lin, in_w, out_w, w_mlp, b_mlp, ln2s, ln2b)


def kernel(x, edge_index_in, edge_index_out, W_lin, b_lin, W_agg_in, b_agg_in,
           W_agg_out, b_agg_out, ln1_scale, ln1_bias, W_mlp, b_mlp, ln2_scale,
           ln2_bias):
    w_agg = jnp.concatenate([W_agg_in, W_agg_out], axis=1)
    b_agg = jnp.concatenate([b_agg_in, b_agg_out]).reshape(1, 2 * M)

    p_in_r, p_out_r = _projections(x, w_agg, b_agg)

    zeros = jnp.zeros((ROWS_PER_TILE, LM), jnp.float32)
    in_coming, out_going = _sc_scatter(
        p_in_r, p_out_r, edge_index_in, edge_index_out, zeros)

    w_mlp2 = ln1_scale[:, None] * W_mlp
    b_mlp2 = (b_mlp + ln1_bias @ W_mlp).reshape(1, OUT)
    return _mlp(x, W_lin, b_lin.reshape(1, OUT), in_coming, out_going,
                w_mlp2, b_mlp2,
                ln2_scale.reshape(1, OUT), ln2_bias.reshape(1, OUT))


# speedup vs baseline: 1.0072x; 1.0072x over previous
import functools

import jax
import jax.numpy as jnp
from jax import lax
from jax.experimental import pallas as pl
from jax.experimental.pallas import tpu as pltpu
from jax.experimental.pallas import tpu_sc as plsc

L = 4
G = 10000
D = 128
OUT = 128
M = 32
E = 320000
CAT = OUT + 2 * M
LM = L * M

NS = 16
CHUNK = 128
NCHUNKS = E // CHUNK
CPT = NCHUNKS // NS
CEXTRA = NCHUNKS - CPT * NS
NRB = 3
NIB = 4
NUNROLL = 12
ROWS_PER_TILE = 624
ROWS_TAIL = G - NS * ROWS_PER_TILE

BG = 2000


def _proj_body(x_ref, w_ref, b_ref, pin_ref, pout_ref):
    w = w_ref[...]
    b = b_ref[...]
    pin_parts = []
    pout_parts = []
    for l in range(L):
        y = jnp.dot(x_ref[l], w, preferred_element_type=jnp.float32) + b
        pin_parts.append(y[:, :M])
        pout_parts.append(y[:, M:])
    pin_ref[...] = jnp.concatenate(pin_parts, axis=1)
    pout_ref[...] = jnp.concatenate(pout_parts, axis=1)


def _projections(x, w_agg, b_agg):
    grid = G // BG
    return pl.pallas_call(
        _proj_body,
        grid=(grid,),
        in_specs=[
            pl.BlockSpec((L, BG, D), lambda i: (0, i, 0)),
            pl.BlockSpec((D, 2 * M), lambda i: (0, 0)),
            pl.BlockSpec((1, 2 * M), lambda i: (0, 0)),
        ],
        out_specs=[
            pl.BlockSpec((BG, LM), lambda i: (i, 0)),
            pl.BlockSpec((BG, LM), lambda i: (i, 0)),
        ],
        out_shape=[
            jax.ShapeDtypeStruct((G, LM), jnp.float32),
            jax.ShapeDtypeStruct((G, LM), jnp.float32),
        ],
    )(x, w_agg, b_agg)


def _sc_scatter(p_in_r, p_out_r, idx_in, idx_out, zeros):
    mesh = plsc.VectorSubcoreMesh(core_axis_name="c", subcore_axis_name="s")

    @functools.partial(
        pl.kernel,
        mesh=mesh,
        out_type=[
            jax.ShapeDtypeStruct((G, LM), jnp.float32),
            jax.ShapeDtypeStruct((G, LM), jnp.float32),
        ],
        scratch_types=(
            [pltpu.VMEM_SHARED((G, LM), jnp.float32)]
            + [pltpu.VMEM((2, CHUNK), jnp.int32) for _ in range(NIB)]
            + [pltpu.VMEM((CHUNK, LM), jnp.float32) for _ in range(NRB)]
            + [pltpu.SemaphoreType.DMA for _ in range(NIB + 2 * NRB)]
        ),
    )
    def k(tab_in, tab_out, idx_i, idx_o, z, out_in, out_out, acc, *bufs):
        ibufs = bufs[:NIB]
        rbufs = bufs[NIB:NIB + NRB]
        si = bufs[NIB + NRB:2 * NIB + NRB]
        sg = bufs[2 * NIB + NRB:2 * NIB + 2 * NRB]
        ss = bufs[2 * NIB + 2 * NRB:]
        c = lax.axis_index("c")
        s = lax.axis_index("s")

        def run(tab, idx_hbm, out_hbm):
            my_rows = pl.ds(s * ROWS_PER_TILE, ROWS_PER_TILE)
            tail_rows = pl.ds(NS * ROWS_PER_TILE, ROWS_TAIL)
            pltpu.sync_copy(z.at[my_rows], acc.at[my_rows])

            @pl.when(s == NS - 1)
            def _():
                pltpu.sync_copy(z.at[tail_rows], acc.at[tail_rows])

            plsc.subcore_barrier()

            cb = s * CPT + jnp.minimum(s, CEXTRA)
            n_chunks = jnp.where(s < CEXTRA, CPT + 1, CPT)

            def fetch_idx(t, bi):
                pltpu.async_copy(idx_hbm.at[cb + t], ibufs[bi], si[bi])

            def wait_idx(bi):
                pltpu.make_async_copy(idx_hbm.at[cb], ibufs[bi], si[bi]).wait()

            def gather(bi, br):
                pltpu.async_copy(tab.at[ibufs[bi].at[1]], rbufs[br], sg[br])

            def wait_gather(bi, br):
                pltpu.make_async_copy(
                    tab.at[ibufs[bi].at[1]], rbufs[br], sg[br]).wait()

            def scatter(bi, br):
                pltpu.async_copy(
                    rbufs[br], acc.at[ibufs[bi].at[0]], ss[br], add=True)

            def wait_scatter(bi, br):
                pltpu.make_async_copy(
                    rbufs[br], acc.at[ibufs[bi].at[0]], ss[br]).wait()

            fetch_idx(jnp.int32(0), 0)
            fetch_idx(jnp.int32(1), 1)
            wait_idx(0)
            gather(0, 0)

            def body(o, carry):
                for off in range(NUNROLL):
                    t = o * NUNROLL + off

                    @pl.when(t >= 2)
                    def _():
                        wait_scatter((off + 2) % NIB, (off + 1) % NRB)

                    @pl.when(t + 2 < n_chunks)
                    def _():
                        fetch_idx(t + 2, (off + 2) % NIB)

                    @pl.when(t + 1 < n_chunks)
                    def _():
                        wait_idx((off + 1) % NIB)
                        gather((off + 1) % NIB, (off + 1) % NRB)

                    wait_gather(off % NIB, off % NRB)
                    scatter(off % NIB, off % NRB)
                return carry

            lax.fori_loop(0, CPT // NUNROLL, body, 0)

            @pl.when(s < CEXTRA)
            def _():
                wait_gather(0, 0)
                scatter(0, 0)

            wait_scatter(2, 1)
            wait_scatter(3, 2)

            @pl.when(s < CEXTRA)
            def _():
                wait_scatter(0, 0)

            plsc.subcore_barrier()
            pltpu.sync_copy(acc.at[my_rows], out_hbm.at[my_rows])

            @pl.when(s == NS - 1)
            def _():
                pltpu.sync_copy(acc.at[tail_rows], out_hbm.at[tail_rows])

        @pl.when(c == 0)
        def _():
            run(tab_in, idx_i, out_in)

        @pl.when(c == 1)
        def _():
            run(tab_out, idx_o, out_out)

    return k(p_in_r, p_out_r, idx_in, idx_out, zeros)


def _silu(v):
    return v / (1.0 + jnp.exp(-v))


def _mlp_body(x_ref, wl_ref, bl_ref, inw_ref, outw_ref,
              wm_ref, bm_ref, ln2s_ref, ln2b_ref, o_ref):
    wl = wl_ref[...]
    bl = bl_ref[...]
    ln2s = ln2s_ref[...]
    ln2b = ln2b_ref[...]
    wm = wm_ref[...]
    bm = bm_ref[...]
    inw = inw_ref[...]
    outw = outw_ref[...]
    for l in range(L):
        proj = jnp.dot(x_ref[l], wl, preferred_element_type=jnp.float32) + bl
        cat = jnp.concatenate(
            [proj, inw[:, l * M:(l + 1) * M], outw[:, l * M:(l + 1) * M]],
            axis=1)
        h = _silu(cat)
        mu = jnp.mean(h, axis=1, keepdims=True)
        msq = jnp.mean(h * h, axis=1, keepdims=True)
        h = (h - mu) * lax.rsqrt(msq - mu * mu + 1e-5)
        h = jnp.dot(h, wm, preferred_element_type=jnp.float32) + bm
        h = _silu(h)
        mu = jnp.mean(h, axis=1, keepdims=True)
        msq = jnp.mean(h * h, axis=1, keepdims=True)
        o_ref[l] = (h - mu) * lax.rsqrt(msq - mu * mu + 1e-5) * ln2s + ln2b


def _mlp(x, w_lin, b_lin, in_w, out_w, w_mlp, b_mlp, ln2s, ln2b):
    grid = G // BG
    return pl.pallas_call(
        _mlp_body,
        grid=(grid,),
        in_specs=[
            pl.BlockSpec((L, BG, D), lambda i: (0, i, 0)),
            pl.BlockSpec((D, OUT), lambda i: (0, 0)),
            pl.BlockSpec((1, OUT), lambda i: (0, 0)),
            pl.BlockSpec((BG, LM), lambda i: (i, 0)),
            pl.BlockSpec((BG, LM), lambda i: (i, 0)),
            pl.BlockSpec((CAT, OUT), lambda i: (0, 0)),
            pl.BlockSpec((1, OUT), lambda i: (0, 0)),
            pl.BlockSpec((1, OUT), lambda i: (0, 0)),
            pl.BlockSpec((1, OUT), lambda i: (0, 0)),
        ],
        out_specs=pl.BlockSpec((L, BG, OUT), lambda i: (0, i, 0)),
        out_shape=jax.ShapeDtypeStruct((L, G, OUT), jnp.float32),
    )(x, w_lin, b_lin, in_w, out_w, w_mlp, b_mlp, ln2s, ln2b)


def kernel(x, edge_index_in, edge_index_out, W_lin, b_lin, W_agg_in, b_agg_in,
           W_agg_out, b_agg_out, ln1_scale, ln1_bias, W_mlp, b_mlp, ln2_scale,
           ln2_bias):
    w_agg = jnp.concatenate([W_agg_in, W_agg_out], axis=1)
    b_agg = jnp.concatenate([b_agg_in, b_agg_out]).reshape(1, 2 * M)

    p_in_r, p_out_r = _projections(x, w_agg, b_agg)

    zeros = jnp.zeros((G, LM), jnp.float32)
    idx_in = jnp.transpose(edge_index_in.reshape(2, NCHUNKS, CHUNK), (1, 0, 2))
    idx_out = jnp.transpose(edge_index_out.reshape(2, NCHUNKS, CHUNK), (1, 0, 2))
    in_coming, out_going = _sc_scatter(p_in_r, p_out_r, idx_in, idx_out, zeros)

    w_mlp2 = ln1_scale[:, None] * W_mlp
    b_mlp2 = (b_mlp + ln1_bias @ W_mlp).reshape(1, OUT)
    return _mlp(x, W_lin, b_lin.reshape(1, OUT), in_coming, out_going,
                w_mlp2, b_mlp2,
                ln2_scale.reshape(1, OUT), ln2_bias.reshape(1, OUT))
